# baseline (speedup 1.0000x reference)
import functools

import jax
import jax.numpy as jnp
from jax import lax
from jax.experimental import pallas as pl
from jax.experimental.pallas import tpu as pltpu
from jax.experimental.pallas import tpu_sc as plsc

B = 16384
D = 64
NC = 2
NS = 16
NW = NC * NS
BPW = B // NW

_mesh = plsc.VectorSubcoreMesh(core_axis_name="c", subcore_axis_name="s")


@functools.partial(
    pl.kernel,
    mesh=_mesh,
    out_type=jax.ShapeDtypeStruct((B, 8, 2 * D), jnp.float32),
    scratch_types=[
        pltpu.VMEM((BPW,), jnp.int32),
        pltpu.VMEM((BPW,), jnp.int32),
        pltpu.VMEM((64, 8, 2 * D), jnp.float32),
        pltpu.SemaphoreType.DMA,
    ],
    compiler_params=pltpu.CompilerParams(use_tc_tiling_on_sc=True),
)
def _block_gather(idx_hbm, tab3_hbm, out_hbm, idx_v, idx16_v, buf, sem):
    wid = lax.axis_index("s") * NC + lax.axis_index("c")
    base = wid * BPW
    pltpu.sync_copy(idx_hbm.at[pl.ds(base, BPW)], idx_v)

    def shift_body(g, carry):
        idx16_v[pl.ds(g * 16, 16)] = lax.shift_right_logical(
            idx_v[pl.ds(g * 16, 16)], 4)
        return carry

    lax.fori_loop(0, BPW // 16, shift_body, 0)

    for k in range(BPW // 64):
        pltpu.async_copy(tab3_hbm.at[idx16_v.at[pl.ds(k * 64, 64)]],
                         buf, sem).wait()
        pltpu.sync_copy(buf, out_hbm.at[pl.ds(base + k * 64, 64)])


@functools.partial(
    pl.kernel,
    mesh=_mesh,
    out_type=jax.ShapeDtypeStruct((B, D), jnp.float32),
    scratch_types=[
        pltpu.VMEM((BPW,), jnp.int32),
        pltpu.VMEM((BPW,), jnp.int32),
        pltpu.VMEM((BPW, D), jnp.float32),
        pltpu.SemaphoreType.DMA,
    ],
    compiler_params=pltpu.CompilerParams(use_tc_tiling_on_sc=False),
)
def _row_select(idx_hbm, scratch_hbm, out_hbm, idx_v, row_v, rows_v, sem):
    wid = lax.axis_index("s") * NC + lax.axis_index("c")
    base = wid * BPW
    pltpu.sync_copy(idx_hbm.at[pl.ds(base, BPW)], idx_v)

    def row_body(g, pos_vec):
        row_v[pl.ds(g * 16, 16)] = pos_vec * 16 + lax.bitwise_and(
            idx_v[pl.ds(g * 16, 16)], 15)
        return pos_vec + 16

    lax.fori_loop(0, BPW // 16, row_body,
                  lax.iota(jnp.int32, 16) + base)

    pltpu.async_copy(scratch_hbm.at[row_v], rows_v, sem).wait()
    pltpu.sync_copy(rows_v, out_hbm.at[pl.ds(base, BPW)])


def kernel(img_ids, codebook):
    idx = img_ids.astype(jnp.int32)
    blocks = _block_gather(idx, codebook.reshape(62500, 8, 2 * D))
    return _row_select(idx, blocks.reshape(B * 16, D))
